# baseline (speedup 1.0000x reference)
import functools

import jax
import jax.numpy as jnp
from jax.experimental import pallas as pl
from jax.experimental.pallas import tpu as pltpu


def _round_up(x: int, m: int) -> int:
    return ((x + m - 1) // m) * m


_BB = 8


def _fused_gcn_kernel(text_ref, adj_ref, w_ref, b_ref, out_ref, *, n):
    j = pl.program_id(1)
    x = text_ref[j]
    h = jax.lax.dot_general(x, w_ref[...], (((1,), (1,)), ((), ())),
                            preferred_element_type=jnp.float32)
    h = h + b_ref[...]
    adj = adj_ref[j]
    agg = jnp.dot(adj, h, preferred_element_type=jnp.float32)
    denom = jnp.sum(adj, axis=1, keepdims=True) + 1.0
    inv = pl.reciprocal(denom, approx=False)
    out_ref[0] = (agg * inv).astype(out_ref.dtype)


def kernel(text, adj, weight, bias):
    B, N, F_in = text.shape
    F_out = weight.shape[0]

    N_pad = _round_up(N, 128)
    F_in_pad = _round_up(F_in, 128)
    F_out_pad = _round_up(F_out, 128)
    bb = _BB if B % _BB == 0 else 1
    B_pad = _round_up(B, bb)

    f32 = jnp.float32
    text_p = jnp.pad(text.astype(f32),
                     ((0, B_pad - B), (0, N_pad - N), (0, F_in_pad - F_in)))
    adj_p = jnp.pad(adj.astype(f32),
                    ((0, B_pad - B), (0, N_pad - N), (0, N_pad - N)))
    w_p = jnp.pad(weight.astype(f32),
                  ((0, F_out_pad - F_out), (0, F_in_pad - F_in)))
    b_p = jnp.pad(bias.astype(f32), (0, F_out_pad - F_out)).reshape(1, -1)

    body = functools.partial(_fused_gcn_kernel, n=N_pad)
    out_p = pl.pallas_call(
        body,
        out_shape=jax.ShapeDtypeStruct((B_pad, N_pad, F_out_pad), text.dtype),
        grid=(B_pad // bb, bb),
        in_specs=[
            pl.BlockSpec((bb, N_pad, F_in_pad), lambda i, j: (i, 0, 0)),
            pl.BlockSpec((bb, N_pad, N_pad), lambda i, j: (i, 0, 0)),
            pl.BlockSpec((F_out_pad, F_in_pad), lambda i, j: (0, 0)),
            pl.BlockSpec((1, F_out_pad), lambda i, j: (0, 0)),
        ],
        out_specs=pl.BlockSpec((1, N_pad, F_out_pad),
                               lambda i, j, bb=bb: (i * bb + j, 0, 0)),
        compiler_params=pltpu.CompilerParams(
            dimension_semantics=("parallel", "arbitrary")),
    )(text_p, adj_p, w_p, b_p)

    return out_p[:B, :N, :F_out]


# speedup vs baseline: 1.9178x; 1.9178x over previous
import functools

import jax
import jax.numpy as jnp
from jax.experimental import pallas as pl
from jax.experimental.pallas import tpu as pltpu


def _round_up(x: int, m: int) -> int:
    return ((x + m - 1) // m) * m


_BB = 8


def _fused_gcn_kernel(text_ref, adj_ref, w_ref, b_ref, out_ref, *, bb, n):
    f_in = w_ref.shape[1]
    x = text_ref[...].reshape(bb * n, f_in)
    h = jax.lax.dot_general(x, w_ref[...], (((1,), (1,)), ((), ())),
                            preferred_element_type=jnp.float32)
    h = h + b_ref[...]
    for i in range(bb):
        adj = adj_ref[i]
        agg = jnp.dot(adj, h[i * n:(i + 1) * n],
                      preferred_element_type=jnp.float32)
        denom = jnp.sum(adj, axis=1, keepdims=True) + 1.0
        inv = pl.reciprocal(denom, approx=False)
        out_ref[i] = (agg * inv).astype(out_ref.dtype)


def kernel(text, adj, weight, bias):
    B, N, F_in = text.shape
    F_out = weight.shape[0]

    N_pad = _round_up(N, 128)
    F_in_pad = _round_up(F_in, 128)
    F_out_pad = _round_up(F_out, 128)
    bb = _BB if B % _BB == 0 else 1
    B_pad = _round_up(B, bb)

    f32 = jnp.float32
    text_p = jnp.pad(text.astype(f32),
                     ((0, B_pad - B), (0, N_pad - N), (0, F_in_pad - F_in)))
    adj_p = jnp.pad(adj.astype(f32),
                    ((0, B_pad - B), (0, N_pad - N), (0, N_pad - N)))
    w_p = jnp.pad(weight.astype(f32),
                  ((0, F_out_pad - F_out), (0, F_in_pad - F_in)))
    b_p = jnp.pad(bias.astype(f32), (0, F_out_pad - F_out)).reshape(1, -1)

    body = functools.partial(_fused_gcn_kernel, bb=bb, n=N_pad)
    out_p = pl.pallas_call(
        body,
        out_shape=jax.ShapeDtypeStruct((B_pad, N_pad, F_out_pad), text.dtype),
        grid=(B_pad // bb,),
        in_specs=[
            pl.BlockSpec((bb, N_pad, F_in_pad), lambda i: (i, 0, 0)),
            pl.BlockSpec((bb, N_pad, N_pad), lambda i: (i, 0, 0)),
            pl.BlockSpec((F_out_pad, F_in_pad), lambda i: (0, 0)),
            pl.BlockSpec((1, F_out_pad), lambda i: (0, 0)),
        ],
        out_specs=pl.BlockSpec((bb, N_pad, F_out_pad), lambda i: (i, 0, 0)),
        compiler_params=pltpu.CompilerParams(
            dimension_semantics=("parallel",)),
    )(text_p, adj_p, w_p, b_p)

    return out_p[:B, :N, :F_out]


# speedup vs baseline: 1.9342x; 1.0086x over previous
import functools

import jax
import jax.numpy as jnp
from jax.experimental import pallas as pl
from jax.experimental.pallas import tpu as pltpu


def _round_up(x: int, m: int) -> int:
    return ((x + m - 1) // m) * m


_BB = 8
_STEPS = 2


def _fused_gcn_kernel(text_ref, adj_ref, w_ref, b_ref, out_ref, h_ref,
                      *, bb, steps, n):
    f_in = w_ref.shape[1]
    j = pl.program_id(1)

    @pl.when(j == 0)
    def _compute_hidden():
        x = text_ref[...].reshape(bb * steps * n, f_in)
        h = jax.lax.dot_general(x, w_ref[...], (((1,), (1,)), ((), ())),
                                preferred_element_type=jnp.float32)
        h_ref[...] = h + b_ref[...]

    for i in range(bb):
        adj = adj_ref[i]
        h_i = h_ref[pl.ds((j * bb + i) * n, n), :]
        agg = jnp.dot(adj, h_i, preferred_element_type=jnp.float32)
        denom = jnp.sum(adj, axis=1, keepdims=True) + 1.0
        inv = pl.reciprocal(denom, approx=False)
        out_ref[i] = (agg * inv).astype(out_ref.dtype)


def kernel(text, adj, weight, bias):
    B, N, F_in = text.shape
    F_out = weight.shape[0]

    N_pad = _round_up(N, 128)
    F_in_pad = _round_up(F_in, 128)
    F_out_pad = _round_up(F_out, 128)
    group = _BB * _STEPS
    if B % group == 0:
        bb, steps = _BB, _STEPS
    else:
        bb, steps = 1, 1
    B_pad = _round_up(B, bb * steps)

    f32 = jnp.float32
    text_p = jnp.pad(text.astype(f32),
                     ((0, B_pad - B), (0, N_pad - N), (0, F_in_pad - F_in)))
    adj_p = jnp.pad(adj.astype(f32),
                    ((0, B_pad - B), (0, N_pad - N), (0, N_pad - N)))
    w_p = jnp.pad(weight.astype(f32),
                  ((0, F_out_pad - F_out), (0, F_in_pad - F_in)))
    b_p = jnp.pad(bias.astype(f32), (0, F_out_pad - F_out)).reshape(1, -1)

    body = functools.partial(_fused_gcn_kernel, bb=bb, steps=steps, n=N_pad)
    out_p = pl.pallas_call(
        body,
        out_shape=jax.ShapeDtypeStruct((B_pad, N_pad, F_out_pad), text.dtype),
        grid=(B_pad // (bb * steps), steps),
        in_specs=[
            pl.BlockSpec((bb * steps, N_pad, F_in_pad),
                         lambda i, j: (i, 0, 0)),
            pl.BlockSpec((bb, N_pad, N_pad),
                         lambda i, j, s=steps: (i * s + j, 0, 0)),
            pl.BlockSpec((F_out_pad, F_in_pad), lambda i, j: (0, 0)),
            pl.BlockSpec((1, F_out_pad), lambda i, j: (0, 0)),
        ],
        out_specs=pl.BlockSpec((bb, N_pad, F_out_pad),
                               lambda i, j, s=steps: (i * s + j, 0, 0)),
        scratch_shapes=[pltpu.VMEM((bb * steps * N_pad, F_out_pad), f32)],
        compiler_params=pltpu.CompilerParams(
            dimension_semantics=("parallel", "arbitrary")),
    )(text_p, adj_p, w_p, b_p)

    return out_p[:B, :N, :F_out]
